# baseline (speedup 1.0000x reference)
import functools

import jax
import jax.numpy as jnp
from jax import lax
from jax.experimental import pallas as pl
from jax.experimental.pallas import tpu as pltpu
from jax.experimental.pallas import tpu_sc as plsc

S = 2048
D = 768
OUT = 768
E = 8
K = 2
TB = 256
NPAIR = S * K
NBUF = NPAIR + E * TB
NBLK = NBUF // TB
NW = 32


def _gate_body(x_ref, gw_ref, gb_ref, i1_ref, i2_ref, p0_ref, p1_ref):
    x = x_ref[...]
    gw = gw_ref[...]
    logits = lax.dot_general(x, gw, (((1,), (1,)), ((), ())),
                             preferred_element_type=jnp.float32)
    logits = logits + gb_ref[...]
    m = jnp.max(logits, axis=1, keepdims=True)
    ex = jnp.exp(logits - m)
    p = ex / jnp.sum(ex, axis=1, keepdims=True)
    ii = lax.broadcasted_iota(jnp.int32, (S, E), 1)
    m1 = jnp.max(p, axis=1, keepdims=True)
    i1 = jnp.min(jnp.where(p == m1, ii, E), axis=1, keepdims=True)
    pm = jnp.where(ii == i1, -1.0, p)
    m2 = jnp.max(pm, axis=1, keepdims=True)
    i2 = jnp.min(jnp.where(pm == m2, ii, E), axis=1, keepdims=True)
    i1_ref[...] = i1
    i2_ref[...] = i2
    p0_ref[...] = jnp.sum(jnp.where(ii == 0, p, 0.0), axis=1, keepdims=True)
    p1_ref[...] = jnp.sum(jnp.where(ii == 1, p, 0.0), axis=1, keepdims=True)


def _gate(x2d, gate_w, gate_b):
    return pl.pallas_call(
        _gate_body,
        out_shape=(
            jax.ShapeDtypeStruct((S, 1), jnp.int32),
            jax.ShapeDtypeStruct((S, 1), jnp.int32),
            jax.ShapeDtypeStruct((S, 1), jnp.float32),
            jax.ShapeDtypeStruct((S, 1), jnp.float32),
        ),
    )(x2d, gate_w, gate_b.reshape(1, E))


_G_PER_W = NBUF // NW
_G_CH = _G_PER_W // 2


@functools.cache
def _sc_mesh():
    return plsc.VectorSubcoreMesh(core_axis_name="c", subcore_axis_name="s")


@functools.cache
def _sc_gather_fn():
    @functools.partial(
        pl.kernel,
        out_type=jax.ShapeDtypeStruct((NBUF, D), jnp.float32),
        mesh=_sc_mesh(),
        scratch_types=[
            pltpu.VMEM((_G_CH,), jnp.int32),
            pltpu.VMEM((_G_CH, D), jnp.float32),
            pltpu.SemaphoreType.DMA,
        ],
    )
    def gather(x_hbm, idx_hbm, out_hbm, idx_v, rows_v, sem):
        wid = lax.axis_index("s") * 2 + lax.axis_index("c")
        for c in range(_G_PER_W // _G_CH):
            base = wid * _G_PER_W + c * _G_CH
            pltpu.sync_copy(idx_hbm.at[pl.ds(base, _G_CH)], idx_v)
            pltpu.async_copy(x_hbm.at[idx_v], rows_v, sem).wait()
            pltpu.sync_copy(rows_v, out_hbm.at[pl.ds(base, _G_CH)])

    return gather


def _sc_gather(x2d, row_token):
    return _sc_gather_fn()(x2d, row_token)


def _mlp_body(be_ref, nv_ref, xg_ref, sc_ref, w1_ref, b1_ref, w2_ref, b2_ref,
              w3_ref, b3_ref, out_ref):
    i = pl.program_id(0)

    @pl.when(i < nv_ref[0])
    def _():
        xb = xg_ref[...]
        h = lax.dot_general(xb, w1_ref[0], (((1,), (1,)), ((), ())),
                            preferred_element_type=jnp.float32)
        h = jnp.maximum(h + b1_ref[0], 0.0)
        h = lax.dot_general(h, w2_ref[0], (((1,), (1,)), ((), ())),
                            preferred_element_type=jnp.float32)
        h = jnp.maximum(h + b2_ref[0], 0.0)
        h = lax.dot_general(h, w3_ref[0], (((1,), (1,)), ((), ())),
                            preferred_element_type=jnp.float32)
        h = h + b3_ref[0]
        out_ref[...] = h * sc_ref[...]


def _mlp(xg, scale, block_expert, nvalid, fc1_w, fc1_b, fc2_w, fc2_b,
         fc3_w, fc3_b):
    grid_spec = pltpu.PrefetchScalarGridSpec(
        num_scalar_prefetch=2,
        grid=(NBLK,),
        in_specs=[
            pl.BlockSpec((TB, D), lambda i, be, nv: (i, 0)),
            pl.BlockSpec((TB, 1), lambda i, be, nv: (i, 0)),
            pl.BlockSpec((1, OUT, D), lambda i, be, nv: (be[i], 0, 0)),
            pl.BlockSpec((1, 1, OUT), lambda i, be, nv: (be[i], 0, 0)),
            pl.BlockSpec((1, OUT, OUT), lambda i, be, nv: (be[i], 0, 0)),
            pl.BlockSpec((1, 1, OUT), lambda i, be, nv: (be[i], 0, 0)),
            pl.BlockSpec((1, OUT, OUT), lambda i, be, nv: (be[i], 0, 0)),
            pl.BlockSpec((1, 1, OUT), lambda i, be, nv: (be[i], 0, 0)),
        ],
        out_specs=pl.BlockSpec((TB, OUT), lambda i, be, nv: (i, 0)),
    )
    return pl.pallas_call(
        _mlp_body,
        grid_spec=grid_spec,
        out_shape=jax.ShapeDtypeStruct((NBUF, OUT), jnp.float32),
    )(block_expert, nvalid, xg, scale, fc1_w, fc1_b.reshape(E, 1, OUT),
      fc2_w, fc2_b.reshape(E, 1, OUT), fc3_w, fc3_b.reshape(E, 1, OUT))


_C_PER_W = S // NW


@functools.cache
def _sc_combine_fn():
    @functools.partial(
        pl.kernel,
        out_type=jax.ShapeDtypeStruct((S, OUT), jnp.float32),
        mesh=_sc_mesh(),
        scratch_types=[
            pltpu.VMEM((_C_PER_W,), jnp.int32),
            pltpu.VMEM((_C_PER_W,), jnp.int32),
            pltpu.VMEM((_C_PER_W, OUT), jnp.float32),
            pltpu.VMEM((_C_PER_W, OUT), jnp.float32),
            pltpu.SemaphoreType.DMA,
            pltpu.SemaphoreType.DMA,
        ],
    )
    def combine(ybuf_hbm, d0_hbm, d1_hbm, out_hbm, i0_v, i1_v, r0_v, r1_v,
                sem0, sem1):
        wid = lax.axis_index("s") * 2 + lax.axis_index("c")
        base = wid * _C_PER_W
        pltpu.sync_copy(d0_hbm.at[pl.ds(base, _C_PER_W)], i0_v)
        pltpu.sync_copy(d1_hbm.at[pl.ds(base, _C_PER_W)], i1_v)
        c0 = pltpu.async_copy(ybuf_hbm.at[i0_v], r0_v, sem0)
        c1 = pltpu.async_copy(ybuf_hbm.at[i1_v], r1_v, sem1)
        c0.wait()
        c1.wait()

        def body(t, carry):
            for j in range(OUT // 16):
                sl = pl.ds(j * 16, 16)
                r0_v[t, sl] = r0_v[t, sl] + r1_v[t, sl]
            return carry

        lax.fori_loop(0, _C_PER_W, body, 0)
        pltpu.sync_copy(r0_v, out_hbm.at[pl.ds(base, _C_PER_W)])

    return combine


def _sc_combine(ybuf, d0, d1):
    return _sc_combine_fn()(ybuf, d0, d1)


def _route(i1, i2, p0, p1):
    ef = jnp.concatenate([i1, i2], axis=1).reshape(-1)
    oh = (ef[:, None] == jnp.arange(E, dtype=jnp.int32)[None, :]).astype(jnp.int32)
    csum = jnp.cumsum(oh, axis=0)
    counts = csum[-1]
    rank = jnp.take_along_axis(csum, ef[:, None], axis=1)[:, 0] - 1
    pc = ((counts + TB - 1) // TB) * TB
    ends = jnp.cumsum(pc)
    starts = ends - pc
    dst = starts[ef] + rank
    row_token = jnp.zeros((NBUF,), jnp.int32).at[dst].set(
        jnp.arange(NPAIR, dtype=jnp.int32) // K)
    pflat = jnp.concatenate([p0, p1], axis=1).reshape(-1)
    scale = jnp.zeros((NBUF, 1), jnp.float32).at[dst, 0].set(pflat)
    nvalid = (ends[-1] // TB).reshape(1).astype(jnp.int32)
    block_expert = jnp.searchsorted(
        ends, jnp.arange(NBLK, dtype=jnp.int32) * TB, side="right")
    block_expert = jnp.minimum(block_expert, E - 1).astype(jnp.int32)
    d0 = dst[0::2]
    d1 = dst[1::2]
    return row_token, scale, block_expert, nvalid, d0, d1


def kernel(x, gate_w, gate_b, fc1_w, fc1_b, fc2_w, fc2_b, fc3_w, fc3_b):
    x2d = x.reshape(S, D)
    i1, i2, p0, p1 = _gate(x2d, gate_w, gate_b)
    row_token, scale, block_expert, nvalid, d0, d1 = _route(i1, i2, p0, p1)
    xg = _sc_gather(x2d, row_token)
    ybuf = _mlp(xg, scale, block_expert, nvalid, fc1_w, fc1_b, fc2_w, fc2_b,
                fc3_w, fc3_b)
    out = _sc_combine(ybuf, d0, d1)
    return out.reshape(1, S, OUT)
